# Initial kernel scaffold; baseline (speedup 1.0000x reference)
import jax
import jax.numpy as jnp
from jax.experimental import pallas as pl


def kernel(token_ids, table, gamma, beta, pe):
    raise NotImplementedError("write your pallas kernel here")


# speedup vs baseline: 1.2193x; 1.2193x over previous
import functools
import math

import jax
import jax.numpy as jnp
from jax import lax
from jax.experimental import pallas as pl
from jax.experimental.pallas import tpu as pltpu
from jax.experimental.pallas import tpu_sc as plsc

H = 128
EPS = 1e-5
NC, NS = 2, 16
NW = NC * NS
CHUNK = 128
NBUF = 4


def _sc_gather(table, ids2d):
    n_rows_chunks = ids2d.shape[0]
    n = n_rows_chunks * CHUNK
    n_chunks = n_rows_chunks // NW
    mesh = plsc.VectorSubcoreMesh(core_axis_name="c", subcore_axis_name="s")

    @functools.partial(
        pl.kernel,
        out_type=jax.ShapeDtypeStruct((n, H), jnp.float32),
        mesh=mesh,
        scratch_types=[
            pltpu.VMEM((n_chunks, CHUNK), jnp.int32),
            *[pltpu.VMEM((CHUNK, H), jnp.float32) for _ in range(NBUF)],
            *[pltpu.SemaphoreType.DMA for _ in range(2 * NBUF)],
        ],
    )
    def gather_kernel(table_hbm, idx_hbm, out_hbm, idx_v, *bufs_and_sems):
        bufs = bufs_and_sems[:NBUF]
        gsems = bufs_and_sems[NBUF:2 * NBUF]
        wsems = bufs_and_sems[2 * NBUF:]
        wid = lax.axis_index("s") * NC + lax.axis_index("c")
        base = wid * n_chunks
        pltpu.sync_copy(idx_hbm.at[pl.ds(base, n_chunks)], idx_v)

        gathers = [None] * n_chunks
        writes = [None] * n_chunks
        gathers[0] = pltpu.async_copy(
            table_hbm.at[idx_v.at[0]], bufs[0], gsems[0])
        for j in range(n_chunks):
            p = j % NBUF
            if j + 1 < n_chunks:
                pn = (j + 1) % NBUF
                if j + 1 >= NBUF:
                    writes[j + 1 - NBUF].wait()
                gathers[j + 1] = pltpu.async_copy(
                    table_hbm.at[idx_v.at[j + 1]], bufs[pn], gsems[pn])
            gathers[j].wait()
            writes[j] = pltpu.async_copy(
                bufs[p], out_hbm.at[pl.ds((base + j) * CHUNK, CHUNK)],
                wsems[p])
        for j in range(max(0, n_chunks - NBUF), n_chunks):
            writes[j].wait()

    return gather_kernel(table, ids2d)


def _ln_body(g_ref, pe_ref, gamma_ref, beta_ref, o_ref):
    x = g_ref[...] * math.sqrt(H) + pe_ref[...]
    mean = jnp.mean(x, axis=-1, keepdims=True)
    xc = x - mean
    var = jnp.mean(xc * xc, axis=-1, keepdims=True)
    o_ref[...] = xc * lax.rsqrt(var + EPS) * gamma_ref[...] + beta_ref[...]


def _layernorm(gathered, pe, gamma, beta, s):
    n = gathered.shape[0]
    blk = 1024
    pe_blocks = s // blk
    return pl.pallas_call(
        _ln_body,
        grid=(n // blk,),
        in_specs=[
            pl.BlockSpec((blk, H), lambda i: (i, 0)),
            pl.BlockSpec((blk, H), lambda i: (i % pe_blocks, 0)),
            pl.BlockSpec((1, H), lambda i: (0, 0)),
            pl.BlockSpec((1, H), lambda i: (0, 0)),
        ],
        out_specs=pl.BlockSpec((blk, H), lambda i: (i, 0)),
        out_shape=jax.ShapeDtypeStruct((n, H), jnp.float32),
    )(gathered, pe, gamma.reshape(1, H), beta.reshape(1, H))


def kernel(token_ids, table, gamma, beta, pe):
    b, s = token_ids.shape
    ids2d = token_ids.reshape(-1, CHUNK).astype(jnp.int32)
    gathered = _sc_gather(table, ids2d)
    out = _layernorm(gathered, pe[:s], gamma, beta, s)
    return out.reshape(b, s, H)
